# baseline (speedup 1.0000x reference)
import math

import jax
import jax.numpy as jnp
from jax.experimental import pallas as pl
from jax.experimental.pallas import tpu as pltpu

_INV_TEMP = 1.0 / math.e
_ACTIVE_RATIO = 0.7


def _router_block(x_ref, wt_ref, o_ref, *, k):
    x = x_ref[...]
    wt = wt_ref[...]
    s = jax.lax.dot_general(
        x, wt,
        (((1,), (0,)), ((), ())),
        preferred_element_type=jnp.float32,
    ) * _INV_TEMP
    gt = (s[:, None, :] > s[:, :, None]).astype(jnp.float32)
    rank = jnp.sum(gt, axis=-1)
    keep = rank < float(k)
    sm = jnp.where(keep, s, -jnp.inf)
    m = jnp.max(sm, axis=-1, keepdims=True)
    p = jnp.exp(sm - m)
    o_ref[...] = p / jnp.sum(p, axis=-1, keepdims=True)


def kernel(x, W):
    B, S, H = x.shape
    E = W.shape[0]
    N = B * S
    k = max(1, int(E * _ACTIVE_RATIO))
    T = 256
    xf = x.reshape(N, H)
    wt = W.T

    import functools
    out = pl.pallas_call(
        functools.partial(_router_block, k=k),
        grid=(N // T,),
        in_specs=[
            pl.BlockSpec((T, H), lambda i: (i, 0)),
            pl.BlockSpec((H, E), lambda i: (0, 0)),
        ],
        out_specs=pl.BlockSpec((T, E), lambda i: (i, 0)),
        out_shape=jax.ShapeDtypeStruct((N, E), jnp.float32),
        compiler_params=pltpu.CompilerParams(
            dimension_semantics=("arbitrary",),
        ),
    )(xf, wt)
    return out.reshape(B, S, E)


# speedup vs baseline: 15.0695x; 15.0695x over previous
import functools
import math

import jax
import jax.numpy as jnp
from jax.experimental import pallas as pl
from jax.experimental.pallas import tpu as pltpu

_INV_TEMP = 1.0 / math.e
_ACTIVE_RATIO = 0.7


def _router_block(x_ref, wt_ref, o_ref, *, n_drop):
    x = x_ref[...]
    wt = wt_ref[...]
    s = jax.lax.dot_general(
        x, wt,
        (((1,), (0,)), ((), ())),
        preferred_element_type=jnp.float32,
    ) * _INV_TEMP
    st = s.T

    def drop_one(_, cur):
        m = jnp.min(cur, axis=0, keepdims=True)
        return jnp.where(cur == m, jnp.inf, cur)

    survived = jax.lax.fori_loop(0, n_drop, drop_one, st)
    keep = survived != jnp.inf

    mx = jnp.max(st, axis=0, keepdims=True)
    p = jnp.where(keep, jnp.exp(st - mx), 0.0)
    w = p / jnp.sum(p, axis=0, keepdims=True)
    o_ref[...] = w.T


def kernel(x, W):
    B, S, H = x.shape
    E = W.shape[0]
    N = B * S
    n_drop = E - max(1, int(E * _ACTIVE_RATIO))
    T = 512
    xf = x.reshape(N, H)
    wt = W.T

    out = pl.pallas_call(
        functools.partial(_router_block, n_drop=n_drop),
        grid=(N // T,),
        in_specs=[
            pl.BlockSpec((T, H), lambda i: (i, 0)),
            pl.BlockSpec((H, E), lambda i: (0, 0)),
        ],
        out_specs=pl.BlockSpec((T, E), lambda i: (i, 0)),
        out_shape=jax.ShapeDtypeStruct((N, E), jnp.float32),
        compiler_params=pltpu.CompilerParams(
            dimension_semantics=("arbitrary",),
        ),
    )(xf, wt)
    return out.reshape(B, S, E)
